# Initial kernel scaffold; baseline (speedup 1.0000x reference)
import jax
import jax.numpy as jnp
from jax.experimental import pallas as pl


def kernel(x, pos, batch, ptr, params):
    raise NotImplementedError("write your pallas kernel here")


# speedup vs baseline: 2.0375x; 2.0375x over previous
import numpy as np
import jax
import jax.numpy as jnp
from jax import lax
from jax.experimental import pallas as pl
from jax.experimental.pallas import tpu as pltpu

_BN = np.float32(1.0 / np.sqrt(1.0 + 1e-6))
_K = 16
_BIG = 2**30


def _leaky(v):
    return jnp.where(v >= 0, v, 0.2 * v)


def _dot(a, b):
    return jnp.dot(a, b, preferred_element_type=jnp.float32)


def _knn_call(pos):
    N = pos.shape[0]
    R = min(64, N)
    C = min(1024, N)
    nch = N // C
    posT = pos.T

    def body(pos_ref, posT_ref, out_ref, cv_ref, ci_ref):
        pr = pos_ref[...]
        r2 = jnp.sum(pr * pr, axis=1, keepdims=True)
        for c in range(nch):
            pcT = posT_ref[:, pl.ds(c * C, C)]
            c2 = jnp.sum(pcT * pcT, axis=0, keepdims=True)
            ab = _dot(pr, pcT)
            d = r2 + c2 - 2.0 * ab
            iot = lax.broadcasted_iota(jnp.int32, (R, C), 1) + (c * C)
            vs, js = [], []
            for _ in range(_K):
                m = jnp.min(d, axis=1, keepdims=True)
                am = jnp.min(jnp.where(d == m, iot, _BIG), axis=1, keepdims=True)
                vs.append(m)
                js.append(am)
                d = jnp.where(iot == am, jnp.float32(np.inf), d)
            cv_ref[:, pl.ds(c * _K, _K)] = jnp.concatenate(vs, axis=1)
            ci_ref[:, pl.ds(c * _K, _K)] = jnp.concatenate(js, axis=1)
        if nch == 1:
            out_ref[...] = ci_ref[...]
        else:
            cv = cv_ref[...]
            ci = ci_ref[...]
            outs = []
            for _ in range(_K):
                m = jnp.min(cv, axis=1, keepdims=True)
                am = jnp.min(jnp.where(cv == m, ci, _BIG), axis=1, keepdims=True)
                outs.append(am)
                cv = jnp.where((cv == m) & (ci == am), jnp.float32(np.inf), cv)
            out_ref[...] = jnp.concatenate(outs, axis=1)

    return pl.pallas_call(
        body,
        grid=(N // R,),
        in_specs=[
            pl.BlockSpec((R, 3), lambda i: (i, 0)),
            pl.BlockSpec((3, N), lambda i: (0, 0)),
        ],
        out_specs=pl.BlockSpec((R, _K), lambda i: (i, 0)),
        out_shape=jax.ShapeDtypeStruct((N, _K), jnp.int32),
        scratch_shapes=[
            pltpu.VMEM((R, _K * nch), jnp.float32),
            pltpu.VMEM((R, _K * nch), jnp.int32),
        ],
    )(pos, posT)


def _pre_call(xin, fc0, sc_p, m1_p):
    N = xin.shape[0]
    R = min(256, N)
    scW, scb = sc_p
    W1, b1 = m1_p
    Cout, Cmid = scW.shape[1], W1.shape[1]
    ws = [scW, scb.reshape(1, -1), W1, b1.reshape(1, -1)]
    if fc0 is not None:
        W0, b0 = fc0
        ws = [W0, b0.reshape(1, -1)] + ws

    def body(x_ref, *refs):
        wr = refs[: len(ws)]
        sc_ref, t_ref = refs[len(ws)], refs[len(ws) + 1]
        h = x_ref[...]
        k = 0
        if fc0 is not None:
            h = _dot(h, wr[0][...]) + wr[1][...]
            k = 2
        sc_ref[...] = (_dot(h, wr[k][...]) + wr[k + 1][...]) * _BN
        t_ref[...] = _leaky((_dot(h, wr[k + 2][...]) + wr[k + 3][...]) * _BN)

    full = lambda a: pl.BlockSpec(a.shape, lambda i: tuple(0 for _ in a.shape))
    return pl.pallas_call(
        body,
        grid=(N // R,),
        in_specs=[pl.BlockSpec((R, xin.shape[1]), lambda i: (i, 0))] + [full(w) for w in ws],
        out_specs=(
            pl.BlockSpec((R, Cout), lambda i: (i, 0)),
            pl.BlockSpec((R, Cmid), lambda i: (i, 0)),
        ),
        out_shape=(
            jax.ShapeDtypeStruct((N, Cout), jnp.float32),
            jax.ShapeDtypeStruct((N, Cmid), jnp.float32),
        ),
    )(xin, *ws)


def _lfa_call(xfeat, pos, nbr_flat, p, ch):
    N = xfeat.shape[0]
    chh = ch // 2
    R = 64
    RK = R * _K
    T = min(1024, N)
    nt = N // T
    encW, encb = p["enc"]
    ws = [encW, encb.reshape(1, -1), p["att"], p["post"][0], p["post"][1].reshape(1, -1)]

    def body(pos_ref, nbrf_ref, xt_ref, pt_ref, encW_r, encb_r, attW_r, postW_r, postb_r, out_ref):
        idxf = nbrf_ref[...]
        pj = jnp.zeros((RK, 3), jnp.float32)
        xj = jnp.zeros((RK, chh), jnp.float32)
        for t in range(nt):
            iot = lax.broadcasted_iota(jnp.int32, (RK, T), 1) + t * T
            oh = (idxf == iot).astype(jnp.float32)
            pj = pj + _dot(oh, pt_ref[pl.ds(t * T, T), :])
            xj = xj + _dot(oh, xt_ref[pl.ds(t * T, T), :])
        pr = pos_ref[...]
        pi = jnp.broadcast_to(pr[:, None, :], (R, _K, 3)).reshape(RK, 3)
        diff = pj - pi
        dist = jnp.sqrt(jnp.sum(diff * diff, axis=1, keepdims=True) + 1e-12)
        rel = jnp.concatenate([pi, pj, diff, dist], axis=1)
        enc = _leaky((_dot(rel, encW_r[...]) + encb_r[...]) * _BN)
        loc = jnp.concatenate([xj, enc], axis=1)
        logit = _dot(loc, attW_r[...])
        l3 = logit.reshape(R, _K, ch)
        mx = jnp.max(l3, axis=(1, 2), keepdims=True)
        e = jnp.exp(l3 - mx)
        s = jnp.sum(e, axis=1, keepdims=True)
        att = e / s
        agg = jnp.sum(att * loc.reshape(R, _K, ch), axis=1)
        out_ref[...] = _leaky((_dot(agg, postW_r[...]) + postb_r[...]) * _BN)

    full = lambda a: pl.BlockSpec(a.shape, lambda i: tuple(0 for _ in a.shape))
    return pl.pallas_call(
        body,
        grid=(N // R,),
        in_specs=[
            pl.BlockSpec((R, 3), lambda i: (i, 0)),
            pl.BlockSpec((RK, 1), lambda i: (i, 0)),
            full(xfeat),
            full(pos),
        ] + [full(w) for w in ws],
        out_specs=pl.BlockSpec((R, ch), lambda i: (i, 0)),
        out_shape=jax.ShapeDtypeStruct((N, ch), jnp.float32),
    )(pos, nbr_flat, xfeat, pos, *ws)


def _post_call(h, sc, m2_p):
    N = h.shape[0]
    R = min(256, N)
    W2, b2 = m2_p
    Cout = W2.shape[1]

    def body(h_ref, sc_ref, W_r, b_r, out_ref):
        out_ref[...] = _leaky((_dot(h_ref[...], W_r[...]) + b_r[...]) * _BN + sc_ref[...])

    full = lambda a: pl.BlockSpec(a.shape, lambda i: tuple(0 for _ in a.shape))
    b2r = b2.reshape(1, -1)
    return pl.pallas_call(
        body,
        grid=(N // R,),
        in_specs=[
            pl.BlockSpec((R, h.shape[1]), lambda i: (i, 0)),
            pl.BlockSpec((R, Cout), lambda i: (i, 0)),
            full(W2),
            full(b2r),
        ],
        out_specs=pl.BlockSpec((R, Cout), lambda i: (i, 0)),
        out_shape=jax.ShapeDtypeStruct((N, Cout), jnp.float32),
    )(h, sc, W2, b2r)


def _fp_call(poss, posc, xcs, xci, xs, psem, pinst, head=None, summit=None):
    Ns, Nc = poss.shape[0], posc.shape[0]
    R = min(64, Ns)
    pcT = posc.T
    Ws_, bs_ = psem
    Wi_, bi_ = pinst
    ws = [Ws_, bs_.reshape(1, -1), Wi_, bi_.reshape(1, -1)]
    if summit is not None:
        (Wsu, bsu), x4d = summit
        ws = [Wsu, bsu.reshape(1, -1), x4d] + ws
        xc_args = []
    else:
        xc_args = [xcs, xci]
    if head is not None:
        (Wc1, bc1), (Wc2, bc2), (Wf, bf) = head
        ws = ws + [Wc1, bc1.reshape(1, -1), Wc2, bc2.reshape(1, -1), Wf, bf.reshape(1, -1)]
        sem_w = Wf.shape[1]
    else:
        sem_w = Ws_.shape[1]

    def body(ps_ref, pcT_ref, xs_ref, *refs):
        n_xc = len(xc_args)
        xc_refs = refs[:n_xc]
        w_refs = refs[n_xc : n_xc + len(ws)]
        sem_ref, inst_ref = refs[-2], refs[-1]
        ps = ps_ref[...]
        pcT = pcT_ref[...]
        r2 = jnp.sum(ps * ps, axis=1, keepdims=True)
        c2 = jnp.sum(pcT * pcT, axis=0, keepdims=True)
        d = r2 + c2 - 2.0 * _dot(ps, pcT)
        iot = lax.broadcasted_iota(jnp.int32, (R, Nc), 1)
        m = jnp.min(d, axis=1, keepdims=True)
        am = jnp.min(jnp.where(d == m, iot, _BIG), axis=1, keepdims=True)
        oh = (iot == am).astype(jnp.float32)
        k = 0
        if summit is not None:
            mm = _leaky((_dot(w_refs[2][...], w_refs[0][...]) + w_refs[1][...]) * _BN)
            xcs_v = xci_v = mm
            k = 3
        else:
            xcs_v, xci_v = xc_refs[0][...], xc_refs[1][...]
        xis = _dot(oh, xcs_v)
        xii = _dot(oh, xci_v)
        xsv = xs_ref[...]
        os_ = _leaky((_dot(jnp.concatenate([xis, xsv], axis=1), w_refs[k][...]) + w_refs[k + 1][...]) * _BN)
        oi_ = _leaky((_dot(jnp.concatenate([xii, xsv], axis=1), w_refs[k + 2][...]) + w_refs[k + 3][...]) * _BN)
        if head is not None:
            hw = w_refs[k + 4 :]
            cc = _leaky((_dot(os_, hw[0][...]) + hw[1][...]) * _BN)
            cc = _leaky((_dot(cc, hw[2][...]) + hw[3][...]) * _BN)
            lg = _dot(cc, hw[4][...]) + hw[5][...]
            mx = jnp.max(lg, axis=1, keepdims=True)
            lse = jnp.log(jnp.sum(jnp.exp(lg - mx), axis=1, keepdims=True))
            sem_ref[...] = lg - mx - lse
        else:
            sem_ref[...] = os_
        inst_ref[...] = oi_

    full = lambda a: pl.BlockSpec(a.shape, lambda i: tuple(0 for _ in a.shape))
    inst_w = Wi_.shape[1]
    return pl.pallas_call(
        body,
        grid=(Ns // R,),
        in_specs=[
            pl.BlockSpec((R, 3), lambda i: (i, 0)),
            pl.BlockSpec((3, Nc), lambda i: (0, 0)),
            pl.BlockSpec((R, xs.shape[1]), lambda i: (i, 0)),
        ] + [full(a) for a in xc_args] + [full(w) for w in ws],
        out_specs=(
            pl.BlockSpec((R, sem_w), lambda i: (i, 0)),
            pl.BlockSpec((R, inst_w), lambda i: (i, 0)),
        ),
        out_shape=(
            jax.ShapeDtypeStruct((Ns, sem_w), jnp.float32),
            jax.ShapeDtypeStruct((Ns, inst_w), jnp.float32),
        ),
    )(poss, pcT, xs, *xc_args, *ws)


def kernel(x, pos, batch, ptr, params):
    p = params
    n = x.shape[0]

    key = jax.random.key(42)
    idx1 = jax.random.permutation(jax.random.fold_in(key, 1), n)[: n // 4]
    idx2 = jax.random.permutation(jax.random.fold_in(key, 2), n // 4)[: n // 16]
    idx3 = jax.random.permutation(jax.random.fold_in(key, 3), n // 16)[: n // 64]
    idx4 = jax.random.permutation(jax.random.fold_in(key, 4), n // 64)[: n // 256]

    def drb(bp, xin, posl, fc0=None):
        N = xin.shape[0]
        nbr = _knn_call(posl)
        nbrf = nbr.reshape(N * _K, 1)
        sc, t = _pre_call(xin, fc0, bp["shortcut"], bp["mlp1"])
        ch1 = 2 * t.shape[1]
        h1 = _lfa_call(t, posl, nbrf, bp["lfa1"], ch1)
        h2 = _lfa_call(h1, posl, nbrf, bp["lfa2"], 2 * ch1)
        return _post_call(h2, sc, bp["mlp2"])

    x1 = drb(p["b1"], x, pos, fc0=p["fc0"])
    x1d, pos1 = x1[idx1], pos[idx1]
    x2 = drb(p["b2"], x1d, pos1)
    x2d, pos2 = x2[idx2], pos1[idx2]
    x3 = drb(p["b3"], x2d, pos2)
    x3d, pos3 = x3[idx3], pos2[idx3]
    x4 = drb(p["b4"], x3d, pos3)
    x4d, pos4 = x4[idx4], pos3[idx4]

    s4, i4 = _fp_call(pos3, pos4, None, None, x3d, p["sem_fp4"], p["inst_fp4"],
                      summit=(p["summit"], x4d))
    s3, i3 = _fp_call(pos2, pos3, s4, i4, x2d, p["sem_fp3"], p["inst_fp3"])
    s2, i2 = _fp_call(pos1, pos2, s3, i3, x1d, p["sem_fp2"], p["inst_fp2"])
    sem, inst = _fp_call(pos, pos1, s2, i2, x1, p["sem_fp1"], p["inst_fp1"],
                         head=(p["classif1"], p["classif2"], p["fc_classif"]))
    return sem, inst, idx4


# speedup vs baseline: 3.2865x; 1.6130x over previous
import numpy as np
import jax
import jax.numpy as jnp
from jax import lax
from jax.experimental import pallas as pl
from jax.experimental.pallas import tpu as pltpu
from jax.experimental.pallas import tpu_sc as plsc

_BN = np.float32(1.0 / np.sqrt(1.0 + 1e-6))
_K = 16
_BIG = 2**30


def _leaky(v):
    return jnp.where(v >= 0, v, 0.2 * v)


def _dot(a, b):
    return jnp.dot(a, b, preferred_element_type=jnp.float32)


def _knn_call(pos):
    N = pos.shape[0]
    R = min(64, N)
    C = min(1024, N)
    nch = N // C
    posT = pos.T

    def body(pos_ref, posT_ref, out_ref, cv_ref, ci_ref):
        pr = pos_ref[...]
        r2 = jnp.sum(pr * pr, axis=1, keepdims=True)
        for c in range(nch):
            pcT = posT_ref[:, pl.ds(c * C, C)]
            c2 = jnp.sum(pcT * pcT, axis=0, keepdims=True)
            ab = _dot(pr, pcT)
            d = r2 + c2 - 2.0 * ab
            iot = lax.broadcasted_iota(jnp.int32, (R, C), 1) + (c * C)
            vs, js = [], []
            for _ in range(_K):
                m = jnp.min(d, axis=1, keepdims=True)
                am = jnp.min(jnp.where(d == m, iot, _BIG), axis=1, keepdims=True)
                vs.append(m)
                js.append(am)
                d = jnp.where(iot == am, jnp.float32(np.inf), d)
            cv_ref[:, pl.ds(c * _K, _K)] = jnp.concatenate(vs, axis=1)
            ci_ref[:, pl.ds(c * _K, _K)] = jnp.concatenate(js, axis=1)
        if nch == 1:
            out_ref[...] = ci_ref[...]
        else:
            cv = cv_ref[...]
            ci = ci_ref[...]
            outs = []
            for _ in range(_K):
                m = jnp.min(cv, axis=1, keepdims=True)
                am = jnp.min(jnp.where(cv == m, ci, _BIG), axis=1, keepdims=True)
                outs.append(am)
                cv = jnp.where((cv == m) & (ci == am), jnp.float32(np.inf), cv)
            out_ref[...] = jnp.concatenate(outs, axis=1)

    return pl.pallas_call(
        body,
        grid=(N // R,),
        in_specs=[
            pl.BlockSpec((R, 3), lambda i: (i, 0)),
            pl.BlockSpec((3, N), lambda i: (0, 0)),
        ],
        out_specs=pl.BlockSpec((R, _K), lambda i: (i, 0)),
        out_shape=jax.ShapeDtypeStruct((N, _K), jnp.int32),
        scratch_shapes=[
            pltpu.VMEM((R, _K * nch), jnp.float32),
            pltpu.VMEM((R, _K * nch), jnp.int32),
        ],
    )(pos, posT)


def _pre_call(xin, fc0, sc_p, m1_p):
    N = xin.shape[0]
    R = min(256, N)
    scW, scb = sc_p
    W1, b1 = m1_p
    Cout, Cmid = scW.shape[1], W1.shape[1]
    ws = [scW, scb.reshape(1, -1), W1, b1.reshape(1, -1)]
    if fc0 is not None:
        W0, b0 = fc0
        ws = [W0, b0.reshape(1, -1)] + ws

    def body(x_ref, *refs):
        wr = refs[: len(ws)]
        sc_ref, t_ref = refs[len(ws)], refs[len(ws) + 1]
        h = x_ref[...]
        k = 0
        if fc0 is not None:
            h = _dot(h, wr[0][...]) + wr[1][...]
            k = 2
        sc_ref[...] = (_dot(h, wr[k][...]) + wr[k + 1][...]) * _BN
        t_ref[...] = _leaky((_dot(h, wr[k + 2][...]) + wr[k + 3][...]) * _BN)

    full = lambda a: pl.BlockSpec(a.shape, lambda i: tuple(0 for _ in a.shape))
    return pl.pallas_call(
        body,
        grid=(N // R,),
        in_specs=[pl.BlockSpec((R, xin.shape[1]), lambda i: (i, 0))] + [full(w) for w in ws],
        out_specs=(
            pl.BlockSpec((R, Cout), lambda i: (i, 0)),
            pl.BlockSpec((R, Cmid), lambda i: (i, 0)),
        ),
        out_shape=(
            jax.ShapeDtypeStruct((N, Cout), jnp.float32),
            jax.ShapeDtypeStruct((N, Cmid), jnp.float32),
        ),
    )(xin, *ws)


def _sc_gather_call(tbl, nbr_flat):
    N, D = tbl.shape
    B = nbr_flat.shape[0]
    NW = 32
    bpw = B // NW
    CH = min(128, bpw)
    nch = bpw // CH
    idx2 = nbr_flat.reshape(B // CH, CH)
    mesh = plsc.VectorSubcoreMesh(core_axis_name="c", subcore_axis_name="s")

    def body(tbl_hbm, idx_hbm, out_hbm, idx_v, rows_v, sem):
        wid = lax.axis_index("s") * 2 + lax.axis_index("c")
        pltpu.sync_copy(idx_hbm.at[pl.ds(wid * nch, nch)], idx_v)

        def fire(j, c):
            pltpu.async_copy(tbl_hbm.at[idx_v.at[j]],
                             rows_v.at[pl.ds(j * CH, CH)], sem)
            return c

        lax.fori_loop(0, nch, fire, 0)
        pltpu.make_async_copy(tbl_hbm.at[pl.ds(0, bpw)], rows_v, sem).wait()
        pltpu.sync_copy(rows_v, out_hbm.at[pl.ds(wid * bpw, bpw)])

    f = pl.kernel(
        body,
        out_type=jax.ShapeDtypeStruct((B, D), jnp.float32),
        mesh=mesh,
        compiler_params=pltpu.CompilerParams(use_tc_tiling_on_sc=False),
        scratch_types=[
            pltpu.VMEM((nch, CH), jnp.int32),
            pltpu.VMEM((bpw, D), jnp.float32),
            pltpu.SemaphoreType.DMA,
        ],
    )
    return f(tbl, idx2)


def _lfa_call(xfeat, pos, nbr_flat, p, ch):
    N = xfeat.shape[0]
    chh = ch // 2
    R = 64
    RK = R * _K
    D = ((chh + 3 + 15) // 16) * 16
    tbl = jnp.pad(jnp.concatenate([xfeat, pos], axis=1), ((0, 0), (0, D - (chh + 3))))
    g = _sc_gather_call(tbl, nbr_flat)
    encW, encb = p["enc"]
    ws = [encW, encb.reshape(1, -1), p["att"], p["post"][0], p["post"][1].reshape(1, -1)]

    def body(pos_ref, g_ref, encW_r, encb_r, attW_r, postW_r, postb_r, out_ref):
        gg = g_ref[...]
        xj = gg[:, :chh]
        pj = gg[:, chh:chh + 3]
        pr = pos_ref[...]
        pi = jnp.broadcast_to(pr[:, None, :], (R, _K, 3)).reshape(RK, 3)
        diff = pj - pi
        dist = jnp.sqrt(jnp.sum(diff * diff, axis=1, keepdims=True) + 1e-12)
        rel = jnp.concatenate([pi, pj, diff, dist], axis=1)
        enc = _leaky((_dot(rel, encW_r[...]) + encb_r[...]) * _BN)
        loc = jnp.concatenate([xj, enc], axis=1)
        logit = _dot(loc, attW_r[...])
        l3 = logit.reshape(R, _K, ch)
        mx = jnp.max(l3, axis=(1, 2), keepdims=True)
        e = jnp.exp(l3 - mx)
        s = jnp.sum(e, axis=1, keepdims=True)
        att = e / s
        agg = jnp.sum(att * loc.reshape(R, _K, ch), axis=1)
        out_ref[...] = _leaky((_dot(agg, postW_r[...]) + postb_r[...]) * _BN)

    full = lambda a: pl.BlockSpec(a.shape, lambda i: tuple(0 for _ in a.shape))
    return pl.pallas_call(
        body,
        grid=(N // R,),
        in_specs=[
            pl.BlockSpec((R, 3), lambda i: (i, 0)),
            pl.BlockSpec((RK, D), lambda i: (i, 0)),
        ] + [full(w) for w in ws],
        out_specs=pl.BlockSpec((R, ch), lambda i: (i, 0)),
        out_shape=jax.ShapeDtypeStruct((N, ch), jnp.float32),
    )(pos, g, *ws)


def _post_call(h, sc, m2_p):
    N = h.shape[0]
    R = min(256, N)
    W2, b2 = m2_p
    Cout = W2.shape[1]

    def body(h_ref, sc_ref, W_r, b_r, out_ref):
        out_ref[...] = _leaky((_dot(h_ref[...], W_r[...]) + b_r[...]) * _BN + sc_ref[...])

    full = lambda a: pl.BlockSpec(a.shape, lambda i: tuple(0 for _ in a.shape))
    b2r = b2.reshape(1, -1)
    return pl.pallas_call(
        body,
        grid=(N // R,),
        in_specs=[
            pl.BlockSpec((R, h.shape[1]), lambda i: (i, 0)),
            pl.BlockSpec((R, Cout), lambda i: (i, 0)),
            full(W2),
            full(b2r),
        ],
        out_specs=pl.BlockSpec((R, Cout), lambda i: (i, 0)),
        out_shape=jax.ShapeDtypeStruct((N, Cout), jnp.float32),
    )(h, sc, W2, b2r)


def _fp_call(poss, posc, xcs, xci, xs, psem, pinst, head=None, summit=None):
    Ns, Nc = poss.shape[0], posc.shape[0]
    R = min(64, Ns)
    pcT = posc.T
    Ws_, bs_ = psem
    Wi_, bi_ = pinst
    ws = [Ws_, bs_.reshape(1, -1), Wi_, bi_.reshape(1, -1)]
    if summit is not None:
        (Wsu, bsu), x4d = summit
        ws = [Wsu, bsu.reshape(1, -1), x4d] + ws
        xc_args = []
    else:
        xc_args = [xcs, xci]
    if head is not None:
        (Wc1, bc1), (Wc2, bc2), (Wf, bf) = head
        ws = ws + [Wc1, bc1.reshape(1, -1), Wc2, bc2.reshape(1, -1), Wf, bf.reshape(1, -1)]
        sem_w = Wf.shape[1]
    else:
        sem_w = Ws_.shape[1]

    def body(ps_ref, pcT_ref, xs_ref, *refs):
        n_xc = len(xc_args)
        xc_refs = refs[:n_xc]
        w_refs = refs[n_xc : n_xc + len(ws)]
        sem_ref, inst_ref = refs[-2], refs[-1]
        ps = ps_ref[...]
        pcT = pcT_ref[...]
        r2 = jnp.sum(ps * ps, axis=1, keepdims=True)
        c2 = jnp.sum(pcT * pcT, axis=0, keepdims=True)
        d = r2 + c2 - 2.0 * _dot(ps, pcT)
        iot = lax.broadcasted_iota(jnp.int32, (R, Nc), 1)
        m = jnp.min(d, axis=1, keepdims=True)
        am = jnp.min(jnp.where(d == m, iot, _BIG), axis=1, keepdims=True)
        oh = (iot == am).astype(jnp.float32)
        k = 0
        if summit is not None:
            mm = _leaky((_dot(w_refs[2][...], w_refs[0][...]) + w_refs[1][...]) * _BN)
            xcs_v = xci_v = mm
            k = 3
        else:
            xcs_v, xci_v = xc_refs[0][...], xc_refs[1][...]
        xis = _dot(oh, xcs_v)
        xii = _dot(oh, xci_v)
        xsv = xs_ref[...]
        os_ = _leaky((_dot(jnp.concatenate([xis, xsv], axis=1), w_refs[k][...]) + w_refs[k + 1][...]) * _BN)
        oi_ = _leaky((_dot(jnp.concatenate([xii, xsv], axis=1), w_refs[k + 2][...]) + w_refs[k + 3][...]) * _BN)
        if head is not None:
            hw = w_refs[k + 4 :]
            cc = _leaky((_dot(os_, hw[0][...]) + hw[1][...]) * _BN)
            cc = _leaky((_dot(cc, hw[2][...]) + hw[3][...]) * _BN)
            lg = _dot(cc, hw[4][...]) + hw[5][...]
            mx = jnp.max(lg, axis=1, keepdims=True)
            lse = jnp.log(jnp.sum(jnp.exp(lg - mx), axis=1, keepdims=True))
            sem_ref[...] = lg - mx - lse
        else:
            sem_ref[...] = os_
        inst_ref[...] = oi_

    full = lambda a: pl.BlockSpec(a.shape, lambda i: tuple(0 for _ in a.shape))
    inst_w = Wi_.shape[1]
    return pl.pallas_call(
        body,
        grid=(Ns // R,),
        in_specs=[
            pl.BlockSpec((R, 3), lambda i: (i, 0)),
            pl.BlockSpec((3, Nc), lambda i: (0, 0)),
            pl.BlockSpec((R, xs.shape[1]), lambda i: (i, 0)),
        ] + [full(a) for a in xc_args] + [full(w) for w in ws],
        out_specs=(
            pl.BlockSpec((R, sem_w), lambda i: (i, 0)),
            pl.BlockSpec((R, inst_w), lambda i: (i, 0)),
        ),
        out_shape=(
            jax.ShapeDtypeStruct((Ns, sem_w), jnp.float32),
            jax.ShapeDtypeStruct((Ns, inst_w), jnp.float32),
        ),
    )(poss, pcT, xs, *xc_args, *ws)


def kernel(x, pos, batch, ptr, params):
    p = params
    n = x.shape[0]

    key = jax.random.key(42)
    idx1 = jax.random.permutation(jax.random.fold_in(key, 1), n)[: n // 4]
    idx2 = jax.random.permutation(jax.random.fold_in(key, 2), n // 4)[: n // 16]
    idx3 = jax.random.permutation(jax.random.fold_in(key, 3), n // 16)[: n // 64]
    idx4 = jax.random.permutation(jax.random.fold_in(key, 4), n // 64)[: n // 256]

    def drb(bp, xin, posl, fc0=None):
        N = xin.shape[0]
        nbr = _knn_call(posl)
        nbrf = nbr.reshape(N * _K)
        sc, t = _pre_call(xin, fc0, bp["shortcut"], bp["mlp1"])
        ch1 = 2 * t.shape[1]
        h1 = _lfa_call(t, posl, nbrf, bp["lfa1"], ch1)
        h2 = _lfa_call(h1, posl, nbrf, bp["lfa2"], 2 * ch1)
        return _post_call(h2, sc, bp["mlp2"])

    x1 = drb(p["b1"], x, pos, fc0=p["fc0"])
    x1d, pos1 = x1[idx1], pos[idx1]
    x2 = drb(p["b2"], x1d, pos1)
    x2d, pos2 = x2[idx2], pos1[idx2]
    x3 = drb(p["b3"], x2d, pos2)
    x3d, pos3 = x3[idx3], pos2[idx3]
    x4 = drb(p["b4"], x3d, pos3)
    x4d, pos4 = x4[idx4], pos3[idx4]

    s4, i4 = _fp_call(pos3, pos4, None, None, x3d, p["sem_fp4"], p["inst_fp4"],
                      summit=(p["summit"], x4d))
    s3, i3 = _fp_call(pos2, pos3, s4, i4, x2d, p["sem_fp3"], p["inst_fp3"])
    s2, i2 = _fp_call(pos1, pos2, s3, i3, x1d, p["sem_fp2"], p["inst_fp2"])
    sem, inst = _fp_call(pos, pos1, s2, i2, x1, p["sem_fp1"], p["inst_fp1"],
                         head=(p["classif1"], p["classif2"], p["fc_classif"]))
    return sem, inst, idx4


# speedup vs baseline: 3.7154x; 1.1305x over previous
import numpy as np
import jax
import jax.numpy as jnp
from jax import lax
from jax.experimental import pallas as pl
from jax.experimental.pallas import tpu as pltpu
from jax.experimental.pallas import tpu_sc as plsc

_BN = np.float32(1.0 / np.sqrt(1.0 + 1e-6))
_K = 16
_BIG = 2**30


def _leaky(v):
    return jnp.where(v >= 0, v, 0.2 * v)


def _dot(a, b):
    return jnp.dot(a, b, preferred_element_type=jnp.float32)


def _knn_call(pos):
    N = pos.shape[0]
    Q = min(128, N)
    C = min(512, N)
    nch = N // C
    posT = pos.T

    def body(posT_ref, pos_ref, out_ref, cv_ref, ci_ref):
        prT = posT_ref[...]
        r2 = jnp.sum(prT * prT, axis=0, keepdims=True)
        s16 = lax.broadcasted_iota(jnp.int32, (_K, Q), 0)

        def chunk(c):
            pc = pos_ref[pl.ds(c * C, C), :]
            c2 = jnp.sum(pc * pc, axis=1, keepdims=True)
            ab = _dot(pc, prT)
            d = c2 + r2 - 2.0 * ab
            iot = lax.broadcasted_iota(jnp.int32, (C, Q), 0) + c * C
            accv = jnp.zeros((_K, Q), jnp.float32)
            acci = jnp.zeros((_K, Q), jnp.int32)
            for t in range(_K):
                m = jnp.min(d, axis=0, keepdims=True)
                am = jnp.min(jnp.where(d == m, iot, _BIG), axis=0, keepdims=True)
                accv = jnp.where(s16 == t, m, accv)
                acci = jnp.where(s16 == t, am, acci)
                d = jnp.where(iot == am, jnp.float32(np.inf), d)
            return accv, acci

        if nch == 1:
            out_ref[...] = chunk(0)[1]
        else:
            def chunk_store(c, carry):
                accv, acci = chunk(c)
                cv_ref[pl.ds(c * _K, _K), :] = accv
                ci_ref[pl.ds(c * _K, _K), :] = acci
                return carry

            lax.fori_loop(0, nch, chunk_store, 0)
        if nch > 1:
            cv = cv_ref[...]
            ci = ci_ref[...]
            acci = jnp.zeros((_K, Q), jnp.int32)
            for t in range(_K):
                m = jnp.min(cv, axis=0, keepdims=True)
                am = jnp.min(jnp.where(cv == m, ci, _BIG), axis=0, keepdims=True)
                acci = jnp.where(s16 == t, am, acci)
                cv = jnp.where((cv == m) & (ci == am), jnp.float32(np.inf), cv)
            out_ref[...] = acci

    return pl.pallas_call(
        body,
        grid=(N // Q,),
        in_specs=[
            pl.BlockSpec((3, Q), lambda i: (0, i)),
            pl.BlockSpec((N, 3), lambda i: (0, 0)),
        ],
        out_specs=pl.BlockSpec((_K, Q), lambda i: (0, i)),
        out_shape=jax.ShapeDtypeStruct((_K, N), jnp.int32),
        scratch_shapes=[
            pltpu.VMEM((_K * nch, Q), jnp.float32),
            pltpu.VMEM((_K * nch, Q), jnp.int32),
        ],
    )(posT, pos)


def _pre_call(xin, fc0, sc_p, m1_p):
    N = xin.shape[0]
    R = min(256, N)
    scW, scb = sc_p
    W1, b1 = m1_p
    Cout, Cmid = scW.shape[1], W1.shape[1]
    ws = [scW, scb.reshape(1, -1), W1, b1.reshape(1, -1)]
    if fc0 is not None:
        W0, b0 = fc0
        ws = [W0, b0.reshape(1, -1)] + ws

    def body(x_ref, *refs):
        wr = refs[: len(ws)]
        sc_ref, t_ref = refs[len(ws)], refs[len(ws) + 1]
        h = x_ref[...]
        k = 0
        if fc0 is not None:
            h = _dot(h, wr[0][...]) + wr[1][...]
            k = 2
        sc_ref[...] = (_dot(h, wr[k][...]) + wr[k + 1][...]) * _BN
        t_ref[...] = _leaky((_dot(h, wr[k + 2][...]) + wr[k + 3][...]) * _BN)

    full = lambda a: pl.BlockSpec(a.shape, lambda i: tuple(0 for _ in a.shape))
    return pl.pallas_call(
        body,
        grid=(N // R,),
        in_specs=[pl.BlockSpec((R, xin.shape[1]), lambda i: (i, 0))] + [full(w) for w in ws],
        out_specs=(
            pl.BlockSpec((R, Cout), lambda i: (i, 0)),
            pl.BlockSpec((R, Cmid), lambda i: (i, 0)),
        ),
        out_shape=(
            jax.ShapeDtypeStruct((N, Cout), jnp.float32),
            jax.ShapeDtypeStruct((N, Cmid), jnp.float32),
        ),
    )(xin, *ws)


def _sc_gather_call(tbl, nbr_flat):
    N, D = tbl.shape
    B = nbr_flat.shape[0]
    NW = 32
    bpw = B // NW
    CH = min(128, bpw)
    nch = bpw // CH
    idx2 = nbr_flat.reshape(B // CH, CH)
    mesh = plsc.VectorSubcoreMesh(core_axis_name="c", subcore_axis_name="s")

    def body(tbl_hbm, idx_hbm, out_hbm, idx_v, rows_v, sem):
        wid = lax.axis_index("s") * 2 + lax.axis_index("c")
        pltpu.sync_copy(idx_hbm.at[pl.ds(wid * nch, nch)], idx_v)

        def fire(j, c):
            pltpu.async_copy(tbl_hbm.at[idx_v.at[j]],
                             rows_v.at[pl.ds(j * CH, CH)], sem)
            return c

        lax.fori_loop(0, nch, fire, 0)
        pltpu.make_async_copy(tbl_hbm.at[pl.ds(0, bpw)], rows_v, sem).wait()
        pltpu.sync_copy(rows_v, out_hbm.at[pl.ds(wid * bpw, bpw)])

    f = pl.kernel(
        body,
        out_type=jax.ShapeDtypeStruct((B, D), jnp.float32),
        mesh=mesh,
        compiler_params=pltpu.CompilerParams(use_tc_tiling_on_sc=False),
        scratch_types=[
            pltpu.VMEM((nch, CH), jnp.int32),
            pltpu.VMEM((bpw, D), jnp.float32),
            pltpu.SemaphoreType.DMA,
        ],
    )
    return f(tbl, idx2)


def _lfa_call(xfeat, pos, nbr_flat, p, ch):
    N = xfeat.shape[0]
    chh = ch // 2
    R = 64
    RK = R * _K
    D = ((chh + 3 + 15) // 16) * 16
    tbl = jnp.pad(jnp.concatenate([xfeat, pos], axis=1), ((0, 0), (0, D - (chh + 3))))
    g = _sc_gather_call(tbl, nbr_flat)
    encW, encb = p["enc"]
    ws = [encW, encb.reshape(1, -1), p["att"], p["post"][0], p["post"][1].reshape(1, -1)]

    def body(pos_ref, g_ref, encW_r, encb_r, attW_r, postW_r, postb_r, out_ref):
        gg = g_ref[...]
        xj = gg[:, :chh]
        pj = gg[:, chh:chh + 3]
        pr = pos_ref[...]
        pi = jnp.broadcast_to(pr[:, None, :], (R, _K, 3)).reshape(RK, 3)
        diff = pj - pi
        dist = jnp.sqrt(jnp.sum(diff * diff, axis=1, keepdims=True) + 1e-12)
        rel = jnp.concatenate([pi, pj, diff, dist], axis=1)
        enc = _leaky((_dot(rel, encW_r[...]) + encb_r[...]) * _BN)
        loc = jnp.concatenate([xj, enc], axis=1)
        logit = _dot(loc, attW_r[...])
        l3 = logit.reshape(R, _K, ch)
        mx = jnp.max(l3, axis=(1, 2), keepdims=True)
        e = jnp.exp(l3 - mx)
        s = jnp.sum(e, axis=1, keepdims=True)
        att = e / s
        agg = jnp.sum(att * loc.reshape(R, _K, ch), axis=1)
        out_ref[...] = _leaky((_dot(agg, postW_r[...]) + postb_r[...]) * _BN)

    full = lambda a: pl.BlockSpec(a.shape, lambda i: tuple(0 for _ in a.shape))
    return pl.pallas_call(
        body,
        grid=(N // R,),
        in_specs=[
            pl.BlockSpec((R, 3), lambda i: (i, 0)),
            pl.BlockSpec((RK, D), lambda i: (i, 0)),
        ] + [full(w) for w in ws],
        out_specs=pl.BlockSpec((R, ch), lambda i: (i, 0)),
        out_shape=jax.ShapeDtypeStruct((N, ch), jnp.float32),
    )(pos, g, *ws)


def _post_call(h, sc, m2_p):
    N = h.shape[0]
    R = min(256, N)
    W2, b2 = m2_p
    Cout = W2.shape[1]

    def body(h_ref, sc_ref, W_r, b_r, out_ref):
        out_ref[...] = _leaky((_dot(h_ref[...], W_r[...]) + b_r[...]) * _BN + sc_ref[...])

    full = lambda a: pl.BlockSpec(a.shape, lambda i: tuple(0 for _ in a.shape))
    b2r = b2.reshape(1, -1)
    return pl.pallas_call(
        body,
        grid=(N // R,),
        in_specs=[
            pl.BlockSpec((R, h.shape[1]), lambda i: (i, 0)),
            pl.BlockSpec((R, Cout), lambda i: (i, 0)),
            full(W2),
            full(b2r),
        ],
        out_specs=pl.BlockSpec((R, Cout), lambda i: (i, 0)),
        out_shape=jax.ShapeDtypeStruct((N, Cout), jnp.float32),
    )(h, sc, W2, b2r)


def _fp_call(poss, posc, xcs, xci, xs, psem, pinst, head=None, summit=None):
    Ns, Nc = poss.shape[0], posc.shape[0]
    R = min(64, Ns)
    pcT = posc.T
    Ws_, bs_ = psem
    Wi_, bi_ = pinst
    ws = [Ws_, bs_.reshape(1, -1), Wi_, bi_.reshape(1, -1)]
    if summit is not None:
        (Wsu, bsu), x4d = summit
        ws = [Wsu, bsu.reshape(1, -1), x4d] + ws
        xc_args = []
    else:
        xc_args = [xcs, xci]
    if head is not None:
        (Wc1, bc1), (Wc2, bc2), (Wf, bf) = head
        ws = ws + [Wc1, bc1.reshape(1, -1), Wc2, bc2.reshape(1, -1), Wf, bf.reshape(1, -1)]
        sem_w = Wf.shape[1]
    else:
        sem_w = Ws_.shape[1]

    def body(ps_ref, pcT_ref, xs_ref, *refs):
        n_xc = len(xc_args)
        xc_refs = refs[:n_xc]
        w_refs = refs[n_xc : n_xc + len(ws)]
        sem_ref, inst_ref = refs[-2], refs[-1]
        ps = ps_ref[...]
        pcT = pcT_ref[...]
        r2 = jnp.sum(ps * ps, axis=1, keepdims=True)
        c2 = jnp.sum(pcT * pcT, axis=0, keepdims=True)
        d = r2 + c2 - 2.0 * _dot(ps, pcT)
        iot = lax.broadcasted_iota(jnp.int32, (R, Nc), 1)
        m = jnp.min(d, axis=1, keepdims=True)
        am = jnp.min(jnp.where(d == m, iot, _BIG), axis=1, keepdims=True)
        oh = (iot == am).astype(jnp.float32)
        k = 0
        if summit is not None:
            mm = _leaky((_dot(w_refs[2][...], w_refs[0][...]) + w_refs[1][...]) * _BN)
            xcs_v = xci_v = mm
            k = 3
        else:
            xcs_v, xci_v = xc_refs[0][...], xc_refs[1][...]
        xis = _dot(oh, xcs_v)
        xii = _dot(oh, xci_v)
        xsv = xs_ref[...]
        os_ = _leaky((_dot(jnp.concatenate([xis, xsv], axis=1), w_refs[k][...]) + w_refs[k + 1][...]) * _BN)
        oi_ = _leaky((_dot(jnp.concatenate([xii, xsv], axis=1), w_refs[k + 2][...]) + w_refs[k + 3][...]) * _BN)
        if head is not None:
            hw = w_refs[k + 4 :]
            cc = _leaky((_dot(os_, hw[0][...]) + hw[1][...]) * _BN)
            cc = _leaky((_dot(cc, hw[2][...]) + hw[3][...]) * _BN)
            lg = _dot(cc, hw[4][...]) + hw[5][...]
            mx = jnp.max(lg, axis=1, keepdims=True)
            lse = jnp.log(jnp.sum(jnp.exp(lg - mx), axis=1, keepdims=True))
            sem_ref[...] = lg - mx - lse
        else:
            sem_ref[...] = os_
        inst_ref[...] = oi_

    full = lambda a: pl.BlockSpec(a.shape, lambda i: tuple(0 for _ in a.shape))
    inst_w = Wi_.shape[1]
    return pl.pallas_call(
        body,
        grid=(Ns // R,),
        in_specs=[
            pl.BlockSpec((R, 3), lambda i: (i, 0)),
            pl.BlockSpec((3, Nc), lambda i: (0, 0)),
            pl.BlockSpec((R, xs.shape[1]), lambda i: (i, 0)),
        ] + [full(a) for a in xc_args] + [full(w) for w in ws],
        out_specs=(
            pl.BlockSpec((R, sem_w), lambda i: (i, 0)),
            pl.BlockSpec((R, inst_w), lambda i: (i, 0)),
        ),
        out_shape=(
            jax.ShapeDtypeStruct((Ns, sem_w), jnp.float32),
            jax.ShapeDtypeStruct((Ns, inst_w), jnp.float32),
        ),
    )(poss, pcT, xs, *xc_args, *ws)


def kernel(x, pos, batch, ptr, params):
    p = params
    n = x.shape[0]

    key = jax.random.key(42)
    idx1 = jax.random.permutation(jax.random.fold_in(key, 1), n)[: n // 4]
    idx2 = jax.random.permutation(jax.random.fold_in(key, 2), n // 4)[: n // 16]
    idx3 = jax.random.permutation(jax.random.fold_in(key, 3), n // 16)[: n // 64]
    idx4 = jax.random.permutation(jax.random.fold_in(key, 4), n // 64)[: n // 256]

    def drb(bp, xin, posl, fc0=None):
        N = xin.shape[0]
        nbrf = _knn_call(posl).T.reshape(N * _K)
        sc, t = _pre_call(xin, fc0, bp["shortcut"], bp["mlp1"])
        ch1 = 2 * t.shape[1]
        h1 = _lfa_call(t, posl, nbrf, bp["lfa1"], ch1)
        h2 = _lfa_call(h1, posl, nbrf, bp["lfa2"], 2 * ch1)
        return _post_call(h2, sc, bp["mlp2"])

    x1 = drb(p["b1"], x, pos, fc0=p["fc0"])
    x1d, pos1 = x1[idx1], pos[idx1]
    x2 = drb(p["b2"], x1d, pos1)
    x2d, pos2 = x2[idx2], pos1[idx2]
    x3 = drb(p["b3"], x2d, pos2)
    x3d, pos3 = x3[idx3], pos2[idx3]
    x4 = drb(p["b4"], x3d, pos3)
    x4d, pos4 = x4[idx4], pos3[idx4]

    s4, i4 = _fp_call(pos3, pos4, None, None, x3d, p["sem_fp4"], p["inst_fp4"],
                      summit=(p["summit"], x4d))
    s3, i3 = _fp_call(pos2, pos3, s4, i4, x2d, p["sem_fp3"], p["inst_fp3"])
    s2, i2 = _fp_call(pos1, pos2, s3, i3, x1d, p["sem_fp2"], p["inst_fp2"])
    sem, inst = _fp_call(pos, pos1, s2, i2, x1, p["sem_fp1"], p["inst_fp1"],
                         head=(p["classif1"], p["classif2"], p["fc_classif"]))
    return sem, inst, idx4


# speedup vs baseline: 4.6589x; 1.2539x over previous
import numpy as np
import jax
import jax.numpy as jnp
from jax import lax
from jax.experimental import pallas as pl
from jax.experimental.pallas import tpu as pltpu
from jax.experimental.pallas import tpu_sc as plsc

_BN = np.float32(1.0 / np.sqrt(1.0 + 1e-6))
_K = 16
_BIG = 2**30


def _leaky(v):
    return jnp.where(v >= 0, v, 0.2 * v)


def _dot(a, b):
    return jnp.dot(a, b, preferred_element_type=jnp.float32)


def _knn_call(pos):
    N = pos.shape[0]
    Q = min(128, N)
    C = min(512, N)
    nch = N // C
    posT = pos.T

    def body(posT_ref, pos_ref, out_ref, ck_ref):
        prT = posT_ref[...]
        r2 = jnp.sum(prT * prT, axis=0, keepdims=True)
        s16 = lax.broadcasted_iota(jnp.int32, (_K, Q), 0)

        def chunk(c):
            pc = pos_ref[pl.ds(c * C, C), :]
            c2 = jnp.sum(pc * pc, axis=1, keepdims=True)
            ab = _dot(pc, prT)
            d = c2 + r2 - 2.0 * ab
            iot = lax.broadcasted_iota(jnp.int32, (C, Q), 0) + c * C
            key = (lax.bitcast_convert_type(d, jnp.int32) & -65536) | iot
            acck = jnp.zeros((_K, Q), jnp.int32)
            for t in range(_K):
                mk = jnp.min(key, axis=0, keepdims=True)
                acck = jnp.where(s16 == t, mk, acck)
                key = jnp.where(key == mk, 2**31 - 1, key)
            return acck

        if nch == 1:
            out_ref[...] = chunk(0) & 65535
        else:
            def chunk_store(c, carry):
                ck_ref[pl.ds(c * _K, _K), :] = chunk(c)
                return carry

            lax.fori_loop(0, nch, chunk_store, 0)
            ck = ck_ref[...]
            acck = jnp.zeros((_K, Q), jnp.int32)
            for t in range(_K):
                mk = jnp.min(ck, axis=0, keepdims=True)
                acck = jnp.where(s16 == t, mk, acck)
                ck = jnp.where(ck == mk, 2**31 - 1, ck)
            out_ref[...] = acck & 65535

    return pl.pallas_call(
        body,
        grid=(N // Q,),
        in_specs=[
            pl.BlockSpec((3, Q), lambda i: (0, i)),
            pl.BlockSpec((N, 3), lambda i: (0, 0)),
        ],
        out_specs=pl.BlockSpec((_K, Q), lambda i: (0, i)),
        out_shape=jax.ShapeDtypeStruct((_K, N), jnp.int32),
        scratch_shapes=[
            pltpu.VMEM((_K * nch, Q), jnp.int32),
        ],
    )(posT, pos)


def _pre_call(xin, fc0, sc_p, m1_p):
    N = xin.shape[0]
    R = min(256, N)
    scW, scb = sc_p
    W1, b1 = m1_p
    Cout, Cmid = scW.shape[1], W1.shape[1]
    ws = [scW, scb.reshape(1, -1), W1, b1.reshape(1, -1)]
    if fc0 is not None:
        W0, b0 = fc0
        ws = [W0, b0.reshape(1, -1)] + ws

    def body(x_ref, *refs):
        wr = refs[: len(ws)]
        sc_ref, t_ref = refs[len(ws)], refs[len(ws) + 1]
        h = x_ref[...]
        k = 0
        if fc0 is not None:
            h = _dot(h, wr[0][...]) + wr[1][...]
            k = 2
        sc_ref[...] = (_dot(h, wr[k][...]) + wr[k + 1][...]) * _BN
        t_ref[...] = _leaky((_dot(h, wr[k + 2][...]) + wr[k + 3][...]) * _BN)

    full = lambda a: pl.BlockSpec(a.shape, lambda i: tuple(0 for _ in a.shape))
    return pl.pallas_call(
        body,
        grid=(N // R,),
        in_specs=[pl.BlockSpec((R, xin.shape[1]), lambda i: (i, 0))] + [full(w) for w in ws],
        out_specs=(
            pl.BlockSpec((R, Cout), lambda i: (i, 0)),
            pl.BlockSpec((R, Cmid), lambda i: (i, 0)),
        ),
        out_shape=(
            jax.ShapeDtypeStruct((N, Cout), jnp.float32),
            jax.ShapeDtypeStruct((N, Cmid), jnp.float32),
        ),
    )(xin, *ws)


def _sc_gather_call(tbl, nbr_flat):
    N, D = tbl.shape
    B = nbr_flat.shape[0]
    NW = 32
    bpw = B // NW
    CH = min(128, bpw)
    nch = bpw // CH
    idx2 = nbr_flat.reshape(B // CH, CH)
    mesh = plsc.VectorSubcoreMesh(core_axis_name="c", subcore_axis_name="s")

    def body(tbl_hbm, idx_hbm, out_hbm, idx_v, rows_v, sem):
        wid = lax.axis_index("s") * 2 + lax.axis_index("c")
        pltpu.sync_copy(idx_hbm.at[pl.ds(wid * nch, nch)], idx_v)

        def fire(j, c):
            pltpu.async_copy(tbl_hbm.at[idx_v.at[j]],
                             rows_v.at[pl.ds(j * CH, CH)], sem)
            return c

        lax.fori_loop(0, nch, fire, 0)
        pltpu.make_async_copy(tbl_hbm.at[pl.ds(0, bpw)], rows_v, sem).wait()
        pltpu.sync_copy(rows_v, out_hbm.at[pl.ds(wid * bpw, bpw)])

    f = pl.kernel(
        body,
        out_type=jax.ShapeDtypeStruct((B, D), jnp.float32),
        mesh=mesh,
        compiler_params=pltpu.CompilerParams(use_tc_tiling_on_sc=False),
        scratch_types=[
            pltpu.VMEM((nch, CH), jnp.int32),
            pltpu.VMEM((bpw, D), jnp.float32),
            pltpu.SemaphoreType.DMA,
        ],
    )
    return f(tbl, idx2)


def _lfa_call(xfeat, pos, nbr_flat, p, ch):
    N = xfeat.shape[0]
    chh = ch // 2
    R = 64
    RK = R * _K
    D = ((chh + 3 + 15) // 16) * 16
    tbl = jnp.pad(jnp.concatenate([xfeat, pos], axis=1), ((0, 0), (0, D - (chh + 3))))
    g = _sc_gather_call(tbl, nbr_flat)
    encW, encb = p["enc"]
    ws = [encW, encb.reshape(1, -1), p["att"], p["post"][0], p["post"][1].reshape(1, -1)]

    def body(pos_ref, g_ref, encW_r, encb_r, attW_r, postW_r, postb_r, out_ref):
        gg = g_ref[...]
        xj = gg[:, :chh]
        pj = gg[:, chh:chh + 3]
        pr = pos_ref[...]
        pi = jnp.broadcast_to(pr[:, None, :], (R, _K, 3)).reshape(RK, 3)
        diff = pj - pi
        dist = jnp.sqrt(jnp.sum(diff * diff, axis=1, keepdims=True) + 1e-12)
        rel = jnp.concatenate([pi, pj, diff, dist], axis=1)
        enc = _leaky((_dot(rel, encW_r[...]) + encb_r[...]) * _BN)
        loc = jnp.concatenate([xj, enc], axis=1)
        logit = _dot(loc, attW_r[...])
        l3 = logit.reshape(R, _K, ch)
        mx = jnp.max(l3, axis=(1, 2), keepdims=True)
        e = jnp.exp(l3 - mx)
        s = jnp.sum(e, axis=1, keepdims=True)
        att = e / s
        agg = jnp.sum(att * loc.reshape(R, _K, ch), axis=1)
        out_ref[...] = _leaky((_dot(agg, postW_r[...]) + postb_r[...]) * _BN)

    full = lambda a: pl.BlockSpec(a.shape, lambda i: tuple(0 for _ in a.shape))
    return pl.pallas_call(
        body,
        grid=(N // R,),
        in_specs=[
            pl.BlockSpec((R, 3), lambda i: (i, 0)),
            pl.BlockSpec((RK, D), lambda i: (i, 0)),
        ] + [full(w) for w in ws],
        out_specs=pl.BlockSpec((R, ch), lambda i: (i, 0)),
        out_shape=jax.ShapeDtypeStruct((N, ch), jnp.float32),
    )(pos, g, *ws)


def _post_call(h, sc, m2_p):
    N = h.shape[0]
    R = min(256, N)
    W2, b2 = m2_p
    Cout = W2.shape[1]

    def body(h_ref, sc_ref, W_r, b_r, out_ref):
        out_ref[...] = _leaky((_dot(h_ref[...], W_r[...]) + b_r[...]) * _BN + sc_ref[...])

    full = lambda a: pl.BlockSpec(a.shape, lambda i: tuple(0 for _ in a.shape))
    b2r = b2.reshape(1, -1)
    return pl.pallas_call(
        body,
        grid=(N // R,),
        in_specs=[
            pl.BlockSpec((R, h.shape[1]), lambda i: (i, 0)),
            pl.BlockSpec((R, Cout), lambda i: (i, 0)),
            full(W2),
            full(b2r),
        ],
        out_specs=pl.BlockSpec((R, Cout), lambda i: (i, 0)),
        out_shape=jax.ShapeDtypeStruct((N, Cout), jnp.float32),
    )(h, sc, W2, b2r)


def _fp_call(poss, posc, xcs, xci, xs, psem, pinst, head=None, summit=None):
    Ns, Nc = poss.shape[0], posc.shape[0]
    R = min(64, Ns)
    pcT = posc.T
    Ws_, bs_ = psem
    Wi_, bi_ = pinst
    ws = [Ws_, bs_.reshape(1, -1), Wi_, bi_.reshape(1, -1)]
    if summit is not None:
        (Wsu, bsu), x4d = summit
        ws = [Wsu, bsu.reshape(1, -1), x4d] + ws
        xc_args = []
    else:
        xc_args = [xcs, xci]
    if head is not None:
        (Wc1, bc1), (Wc2, bc2), (Wf, bf) = head
        ws = ws + [Wc1, bc1.reshape(1, -1), Wc2, bc2.reshape(1, -1), Wf, bf.reshape(1, -1)]
        sem_w = Wf.shape[1]
    else:
        sem_w = Ws_.shape[1]

    def body(ps_ref, pcT_ref, xs_ref, *refs):
        n_xc = len(xc_args)
        xc_refs = refs[:n_xc]
        w_refs = refs[n_xc : n_xc + len(ws)]
        sem_ref, inst_ref = refs[-2], refs[-1]
        ps = ps_ref[...]
        pcT = pcT_ref[...]
        r2 = jnp.sum(ps * ps, axis=1, keepdims=True)
        c2 = jnp.sum(pcT * pcT, axis=0, keepdims=True)
        d = r2 + c2 - 2.0 * _dot(ps, pcT)
        iot = lax.broadcasted_iota(jnp.int32, (R, Nc), 1)
        m = jnp.min(d, axis=1, keepdims=True)
        am = jnp.min(jnp.where(d == m, iot, _BIG), axis=1, keepdims=True)
        oh = (iot == am).astype(jnp.float32)
        k = 0
        if summit is not None:
            mm = _leaky((_dot(w_refs[2][...], w_refs[0][...]) + w_refs[1][...]) * _BN)
            xcs_v = xci_v = mm
            k = 3
        else:
            xcs_v, xci_v = xc_refs[0][...], xc_refs[1][...]
        xis = _dot(oh, xcs_v)
        xii = _dot(oh, xci_v)
        xsv = xs_ref[...]
        os_ = _leaky((_dot(jnp.concatenate([xis, xsv], axis=1), w_refs[k][...]) + w_refs[k + 1][...]) * _BN)
        oi_ = _leaky((_dot(jnp.concatenate([xii, xsv], axis=1), w_refs[k + 2][...]) + w_refs[k + 3][...]) * _BN)
        if head is not None:
            hw = w_refs[k + 4 :]
            cc = _leaky((_dot(os_, hw[0][...]) + hw[1][...]) * _BN)
            cc = _leaky((_dot(cc, hw[2][...]) + hw[3][...]) * _BN)
            lg = _dot(cc, hw[4][...]) + hw[5][...]
            mx = jnp.max(lg, axis=1, keepdims=True)
            lse = jnp.log(jnp.sum(jnp.exp(lg - mx), axis=1, keepdims=True))
            sem_ref[...] = lg - mx - lse
        else:
            sem_ref[...] = os_
        inst_ref[...] = oi_

    full = lambda a: pl.BlockSpec(a.shape, lambda i: tuple(0 for _ in a.shape))
    inst_w = Wi_.shape[1]
    return pl.pallas_call(
        body,
        grid=(Ns // R,),
        in_specs=[
            pl.BlockSpec((R, 3), lambda i: (i, 0)),
            pl.BlockSpec((3, Nc), lambda i: (0, 0)),
            pl.BlockSpec((R, xs.shape[1]), lambda i: (i, 0)),
        ] + [full(a) for a in xc_args] + [full(w) for w in ws],
        out_specs=(
            pl.BlockSpec((R, sem_w), lambda i: (i, 0)),
            pl.BlockSpec((R, inst_w), lambda i: (i, 0)),
        ),
        out_shape=(
            jax.ShapeDtypeStruct((Ns, sem_w), jnp.float32),
            jax.ShapeDtypeStruct((Ns, inst_w), jnp.float32),
        ),
    )(poss, pcT, xs, *xc_args, *ws)


def kernel(x, pos, batch, ptr, params):
    p = params
    n = x.shape[0]

    key = jax.random.key(42)
    idx1 = jax.random.permutation(jax.random.fold_in(key, 1), n)[: n // 4]
    idx2 = jax.random.permutation(jax.random.fold_in(key, 2), n // 4)[: n // 16]
    idx3 = jax.random.permutation(jax.random.fold_in(key, 3), n // 16)[: n // 64]
    idx4 = jax.random.permutation(jax.random.fold_in(key, 4), n // 64)[: n // 256]

    def drb(bp, xin, posl, fc0=None):
        N = xin.shape[0]
        nbrf = _knn_call(posl).T.reshape(N * _K)
        sc, t = _pre_call(xin, fc0, bp["shortcut"], bp["mlp1"])
        ch1 = 2 * t.shape[1]
        h1 = _lfa_call(t, posl, nbrf, bp["lfa1"], ch1)
        h2 = _lfa_call(h1, posl, nbrf, bp["lfa2"], 2 * ch1)
        return _post_call(h2, sc, bp["mlp2"])

    x1 = drb(p["b1"], x, pos, fc0=p["fc0"])
    x1d, pos1 = x1[idx1], pos[idx1]
    x2 = drb(p["b2"], x1d, pos1)
    x2d, pos2 = x2[idx2], pos1[idx2]
    x3 = drb(p["b3"], x2d, pos2)
    x3d, pos3 = x3[idx3], pos2[idx3]
    x4 = drb(p["b4"], x3d, pos3)
    x4d, pos4 = x4[idx4], pos3[idx4]

    s4, i4 = _fp_call(pos3, pos4, None, None, x3d, p["sem_fp4"], p["inst_fp4"],
                      summit=(p["summit"], x4d))
    s3, i3 = _fp_call(pos2, pos3, s4, i4, x2d, p["sem_fp3"], p["inst_fp3"])
    s2, i2 = _fp_call(pos1, pos2, s3, i3, x1d, p["sem_fp2"], p["inst_fp2"])
    sem, inst = _fp_call(pos, pos1, s2, i2, x1, p["sem_fp1"], p["inst_fp1"],
                         head=(p["classif1"], p["classif2"], p["fc_classif"]))
    return sem, inst, idx4
